# baseline (speedup 1.0000x reference)
import functools

import jax
import jax.numpy as jnp
from jax import lax
from jax.experimental import pallas as pl
from jax.experimental.pallas import tpu as pltpu

_VMEM_LIMIT = 48 * 1024 * 1024


def _sgta_one(x, w3_ref, projw_ref, trow_ref, *, dim, num_heads, h, w):
    c_head = dim // num_heads
    n = h * w

    pos = lax.broadcasted_iota(jnp.int32, (1, n), 1)
    py = pos // w
    px = pos % w

    taps = []
    for dy in (-1, 0, 1):
        for dx in (-1, 0, 1):
            off = dy * w + dx
            shifted = x if off == 0 else jnp.roll(x, shift=-off, axis=1)
            if dy == 0 and dx == 0:
                xt = shifted
            else:
                valid = ((py + dy >= 0) & (py + dy < h) &
                         (px + dx >= 0) & (px + dx < w))
                xt = jnp.where(valid, shifted, jnp.zeros((), x.dtype))
            taps.append(xt)
    xstack = jnp.concatenate(taps, axis=0)
    qkv = jnp.dot(w3_ref[...], xstack,
                  preferred_element_type=jnp.float32)

    q = qkv[0 * dim:1 * dim]
    k = qkv[1 * dim:2 * dim]
    v = qkv[2 * dim:3 * dim]

    inv_eps = jnp.float32(1e12)
    qn = q * jnp.minimum(lax.rsqrt(jnp.sum(q * q, axis=-1, keepdims=True)),
                         inv_eps)
    kn = k * jnp.minimum(lax.rsqrt(jnp.sum(k * k, axis=-1, keepdims=True)),
                         inv_eps)

    gram = lax.dot_general(qn.astype(jnp.bfloat16), kn.astype(jnp.bfloat16),
                           (((1,), (1,)), ((), ())),
                           preferred_element_type=jnp.float32)
    gram = gram * trow_ref[...]

    row_head = lax.broadcasted_iota(jnp.int32, (dim, dim), 0) // c_head
    col_head = lax.broadcasted_iota(jnp.int32, (dim, dim), 1) // c_head
    gram = jnp.where(row_head == col_head, gram, jnp.float32(-1e30))

    gram = gram - jnp.max(gram, axis=-1, keepdims=True)
    p = jnp.exp(gram)
    p = p * pl.reciprocal(jnp.sum(p, axis=-1, keepdims=True), approx=True)

    ctx = jnp.dot(p.astype(jnp.bfloat16), v.astype(jnp.bfloat16),
                  preferred_element_type=jnp.float32)
    out = jnp.dot(projw_ref[...], ctx.astype(jnp.bfloat16),
                  preferred_element_type=jnp.float32)
    return out


def _sgta_kernel(x_ref, w3_ref, projw_ref, trow_ref, o_ref,
                 *, nb, dim, num_heads, h, w):
    for sb in range(nb):
        out = _sgta_one(x_ref[sb].astype(jnp.bfloat16), w3_ref, projw_ref,
                        trow_ref, dim=dim, num_heads=num_heads, h=h, w=w)
        o_ref[sb] = out.astype(o_ref.dtype)


def kernel(x, qkv_w, qkv_dw_w, proj_w, temperature):
    b, c, h, w = x.shape
    n = h * w
    num_heads = temperature.size
    c_head = c // num_heads
    c3 = 3 * c

    x_cn = x.reshape(b, c, n)

    dww = qkv_dw_w.reshape(c3, 9)
    w3 = (dww[:, :, None] * qkv_w[:, None, :]).reshape(c3, 9 * c)
    w3 = w3.astype(jnp.bfloat16)
    trow = jnp.repeat(temperature.reshape(-1).astype(jnp.float32),
                      c_head).reshape(c, 1)

    nb = 4 if b % 4 == 0 else 1
    body = functools.partial(_sgta_kernel, nb=nb, dim=c,
                             num_heads=num_heads, h=h, w=w)
    out = pl.pallas_call(
        body,
        out_shape=jax.ShapeDtypeStruct((b, c, n), x.dtype),
        grid=(b // nb,),
        in_specs=[
            pl.BlockSpec((nb, c, n), lambda bi: (bi, 0, 0)),
            pl.BlockSpec((c3, 9 * c), lambda bi: (0, 0)),
            pl.BlockSpec((c, c), lambda bi: (0, 0)),
            pl.BlockSpec((c, 1), lambda bi: (0, 0)),
        ],
        out_specs=pl.BlockSpec((nb, c, n), lambda bi: (bi, 0, 0)),
        compiler_params=pltpu.CompilerParams(
            dimension_semantics=("parallel",),
            vmem_limit_bytes=_VMEM_LIMIT),
    )(x_cn, w3, proj_w.astype(jnp.bfloat16), trow)
    return out.reshape(b, c, h, w)


# speedup vs baseline: 1.0308x; 1.0308x over previous
import functools

import jax
import jax.numpy as jnp
from jax import lax
from jax.experimental import pallas as pl
from jax.experimental.pallas import tpu as pltpu

_VMEM_LIMIT = 48 * 1024 * 1024


def _sgta_one(x, w3, projw_ref, trow_ref, *, dim, num_heads, h, w):
    c_head = dim // num_heads
    n = h * w

    pos = lax.broadcasted_iota(jnp.int32, (1, n), 1)
    py = pos // w
    px = pos % w

    taps = []
    for dy in (-1, 0, 1):
        for dx in (-1, 0, 1):
            off = dy * w + dx
            shifted = x if off == 0 else jnp.roll(x, shift=-off, axis=1)
            if dy == 0 and dx == 0:
                xt = shifted
            else:
                valid = ((py + dy >= 0) & (py + dy < h) &
                         (px + dx >= 0) & (px + dx < w))
                xt = jnp.where(valid, shifted, jnp.zeros((), x.dtype))
            taps.append(xt)
    xstack = jnp.concatenate(taps, axis=0)
    qkv = jnp.dot(w3, xstack,
                  preferred_element_type=jnp.float32)

    q = qkv[0 * dim:1 * dim]
    k = qkv[1 * dim:2 * dim]
    v = qkv[2 * dim:3 * dim]

    inv_eps = jnp.float32(1e12)
    qn = q * jnp.minimum(lax.rsqrt(jnp.sum(q * q, axis=-1, keepdims=True)),
                         inv_eps)
    kn = k * jnp.minimum(lax.rsqrt(jnp.sum(k * k, axis=-1, keepdims=True)),
                         inv_eps)

    gram = lax.dot_general(qn.astype(jnp.bfloat16), kn.astype(jnp.bfloat16),
                           (((1,), (1,)), ((), ())),
                           preferred_element_type=jnp.float32)
    gram = gram * trow_ref[...]

    row_head = lax.broadcasted_iota(jnp.int32, (dim, dim), 0) // c_head
    col_head = lax.broadcasted_iota(jnp.int32, (dim, dim), 1) // c_head
    gram = jnp.where(row_head == col_head, gram, jnp.float32(-1e30))

    gram = gram - jnp.max(gram, axis=-1, keepdims=True)
    p = jnp.exp(gram)
    p = p * pl.reciprocal(jnp.sum(p, axis=-1, keepdims=True), approx=True)

    ctx = jnp.dot(p.astype(jnp.bfloat16), v.astype(jnp.bfloat16),
                  preferred_element_type=jnp.float32)
    out = jnp.dot(projw_ref[...], ctx.astype(jnp.bfloat16),
                  preferred_element_type=jnp.float32)
    return out


def _sgta_kernel(qkvw_ref, dww_ref, x_ref, projw_ref, trow_ref, o_ref,
                 w3_ref, *, nb, dim, num_heads, h, w):
    @pl.when(pl.program_id(0) == 0)
    def _():
        qkvw = qkvw_ref[...]
        for tap in range(9):
            w3_ref[:, tap * dim:(tap + 1) * dim] = (
                dww_ref[:, tap:tap + 1] * qkvw).astype(w3_ref.dtype)

    w3 = w3_ref[...]
    for sb in range(nb):
        out = _sgta_one(x_ref[sb].astype(jnp.bfloat16), w3, projw_ref,
                        trow_ref, dim=dim, num_heads=num_heads, h=h, w=w)
        o_ref[sb] = out.astype(o_ref.dtype)


def kernel(x, qkv_w, qkv_dw_w, proj_w, temperature):
    b, c, h, w = x.shape
    n = h * w
    num_heads = temperature.size
    c_head = c // num_heads
    c3 = 3 * c

    x_cn = x.reshape(b, c, n)

    dww = qkv_dw_w.reshape(c3, 9)
    trow = jnp.repeat(temperature.reshape(-1).astype(jnp.float32),
                      c_head).reshape(c, 1)

    nb = 4 if b % 4 == 0 else 1
    body = functools.partial(_sgta_kernel, nb=nb, dim=c,
                             num_heads=num_heads, h=h, w=w)
    out = pl.pallas_call(
        body,
        out_shape=jax.ShapeDtypeStruct((b, c, n), x.dtype),
        grid=(b // nb,),
        in_specs=[
            pl.BlockSpec((c3, c), lambda bi: (0, 0)),
            pl.BlockSpec((c3, 9), lambda bi: (0, 0)),
            pl.BlockSpec((nb, c, n), lambda bi: (bi, 0, 0)),
            pl.BlockSpec((c, c), lambda bi: (0, 0)),
            pl.BlockSpec((c, 1), lambda bi: (0, 0)),
        ],
        out_specs=pl.BlockSpec((nb, c, n), lambda bi: (bi, 0, 0)),
        scratch_shapes=[pltpu.VMEM((c3, 9 * c), jnp.bfloat16)],
        compiler_params=pltpu.CompilerParams(
            dimension_semantics=("arbitrary",),
            vmem_limit_bytes=_VMEM_LIMIT),
    )(qkv_w, dww, x_cn, proj_w.astype(jnp.bfloat16), trow)
    return out.reshape(b, c, h, w)


# speedup vs baseline: 1.0564x; 1.0248x over previous
import functools

import jax
import jax.numpy as jnp
from jax import lax
from jax.experimental import pallas as pl
from jax.experimental.pallas import tpu as pltpu

_VMEM_LIMIT = 48 * 1024 * 1024


def _prep_taps(x, *, h, w):
    n_total = x.shape[-1]
    pos = lax.broadcasted_iota(jnp.int32, (1, n_total), 1) % (h * w)
    py = pos // w
    px = pos % w
    zero = jnp.zeros((), x.dtype)
    bases = {}
    for dx in (-1, 0, 1):
        bdx = x if dx == 0 else jnp.roll(x, shift=-dx, axis=1)
        if dx != 0:
            bdx = jnp.where((px + dx >= 0) & (px + dx < w), bdx, zero)
        bases[dx] = bdx
    taps = []
    for dy in (-1, 0, 1):
        for dx in (-1, 0, 1):
            bdx = bases[dx]
            xt = bdx if dy == 0 else jnp.roll(bdx, shift=-dy * w, axis=1)
            if dy != 0:
                xt = jnp.where((py + dy >= 0) & (py + dy < h), xt, zero)
            taps.append(xt)
    return jnp.concatenate(taps, axis=0)


def _sgta_tail(qkv, projw_ref, trow_ref, *, dim, num_heads):
    c_head = dim // num_heads

    q = qkv[0 * dim:1 * dim]
    k = qkv[1 * dim:2 * dim]
    v = qkv[2 * dim:3 * dim]

    inv_eps = jnp.float32(1e12)
    qn = q * jnp.minimum(lax.rsqrt(jnp.sum(q * q, axis=-1, keepdims=True)),
                         inv_eps)
    kn = k * jnp.minimum(lax.rsqrt(jnp.sum(k * k, axis=-1, keepdims=True)),
                         inv_eps)

    gram = lax.dot_general(qn.astype(jnp.bfloat16), kn.astype(jnp.bfloat16),
                           (((1,), (1,)), ((), ())),
                           preferred_element_type=jnp.float32)
    gram = gram * trow_ref[...]

    row_head = lax.broadcasted_iota(jnp.int32, (dim, dim), 0) // c_head
    col_head = lax.broadcasted_iota(jnp.int32, (dim, dim), 1) // c_head
    gram = jnp.where(row_head == col_head, gram, jnp.float32(-1e30))

    gram = gram - jnp.max(gram, axis=-1, keepdims=True)
    p = jnp.exp(gram)
    p = p * pl.reciprocal(jnp.sum(p, axis=-1, keepdims=True), approx=True)

    ctx = jnp.dot(p.astype(jnp.bfloat16), v.astype(jnp.bfloat16),
                  preferred_element_type=jnp.float32)
    out = jnp.dot(projw_ref[...], ctx.astype(jnp.bfloat16),
                  preferred_element_type=jnp.float32)
    return out


def _sgta_kernel(qkvw_ref, dww_ref, x_ref, projw_ref, trow_ref, o_ref,
                 w3_ref, *, nb, dim, num_heads, h, w):
    @pl.when(pl.program_id(0) == 0)
    def _():
        qkvw = qkvw_ref[...]
        for tap in range(9):
            w3_ref[:, tap * dim:(tap + 1) * dim] = (
                dww_ref[:, tap:tap + 1] * qkvw).astype(w3_ref.dtype)

    w3 = w3_ref[...]
    n = h * w
    for pb in range(nb // 2):
        x2 = jnp.concatenate(
            [x_ref[2 * pb].astype(jnp.bfloat16),
             x_ref[2 * pb + 1].astype(jnp.bfloat16)], axis=1)
        xstack2 = _prep_taps(x2, h=h, w=w)
        qkv2 = jnp.dot(w3, xstack2,
                       preferred_element_type=jnp.float32)
        for i in range(2):
            out = _sgta_tail(qkv2[:, i * n:(i + 1) * n], projw_ref,
                             trow_ref, dim=dim, num_heads=num_heads)
            o_ref[2 * pb + i] = out.astype(o_ref.dtype)
    if nb % 2:
        qkv = jnp.dot(w3, _prep_taps(x_ref[nb - 1].astype(jnp.bfloat16),
                                     h=h, w=w),
                      preferred_element_type=jnp.float32)
        out = _sgta_tail(qkv, projw_ref, trow_ref,
                         dim=dim, num_heads=num_heads)
        o_ref[nb - 1] = out.astype(o_ref.dtype)


def kernel(x, qkv_w, qkv_dw_w, proj_w, temperature):
    b, c, h, w = x.shape
    n = h * w
    num_heads = temperature.size
    c_head = c // num_heads
    c3 = 3 * c

    x_cn = x.reshape(b, c, n)

    dww = qkv_dw_w.reshape(c3, 9)
    trow = jnp.repeat(temperature.reshape(-1).astype(jnp.float32),
                      c_head).reshape(c, 1)

    nb = 4 if b % 4 == 0 else 1
    body = functools.partial(_sgta_kernel, nb=nb, dim=c,
                             num_heads=num_heads, h=h, w=w)
    out = pl.pallas_call(
        body,
        out_shape=jax.ShapeDtypeStruct((b, c, n), x.dtype),
        grid=(b // nb,),
        in_specs=[
            pl.BlockSpec((c3, c), lambda bi: (0, 0)),
            pl.BlockSpec((c3, 9), lambda bi: (0, 0)),
            pl.BlockSpec((nb, c, n), lambda bi: (bi, 0, 0)),
            pl.BlockSpec((c, c), lambda bi: (0, 0)),
            pl.BlockSpec((c, 1), lambda bi: (0, 0)),
        ],
        out_specs=pl.BlockSpec((nb, c, n), lambda bi: (bi, 0, 0)),
        scratch_shapes=[pltpu.VMEM((c3, 9 * c), jnp.bfloat16)],
        compiler_params=pltpu.CompilerParams(
            dimension_semantics=("arbitrary",),
            vmem_limit_bytes=_VMEM_LIMIT),
    )(qkv_w, dww, x_cn, proj_w.astype(jnp.bfloat16), trow)
    return out.reshape(b, c, h, w)


# speedup vs baseline: 1.0951x; 1.0367x over previous
import functools

import jax
import jax.numpy as jnp
from jax import lax
from jax.experimental import pallas as pl
from jax.experimental.pallas import tpu as pltpu

_VMEM_LIMIT = 48 * 1024 * 1024


def _prep_taps(x, *, h, w):
    n_total = x.shape[-1]
    pos = lax.broadcasted_iota(jnp.int32, (1, n_total), 1) % (h * w)
    py = pos // w
    px = pos % w
    zero = jnp.zeros((), x.dtype)
    bases = {}
    for dx in (-1, 0, 1):
        bdx = x if dx == 0 else jnp.roll(x, shift=-dx, axis=1)
        if dx != 0:
            bdx = jnp.where((px + dx >= 0) & (px + dx < w), bdx, zero)
        bases[dx] = bdx
    taps = []
    for dy in (-1, 0, 1):
        for dx in (-1, 0, 1):
            bdx = bases[dx]
            xt = bdx if dy == 0 else jnp.roll(bdx, shift=-dy * w, axis=1)
            if dy != 0:
                xt = jnp.where((py + dy >= 0) & (py + dy < h), xt, zero)
            taps.append(xt)
    return jnp.concatenate(taps, axis=0)


def _sgta_tails(qkvs, projw_ref, trow_ref, *, dim, num_heads):
    c_head = dim // num_heads
    inv_eps = jnp.float32(1e12)

    qns, kns, vs = [], [], []
    for qkv in qkvs:
        q = qkv[0 * dim:1 * dim]
        k = qkv[1 * dim:2 * dim]
        vs.append(qkv[2 * dim:3 * dim])
        qn = q * jnp.minimum(
            lax.rsqrt(jnp.sum(q * q, axis=-1, keepdims=True)), inv_eps)
        kn = k * jnp.minimum(
            lax.rsqrt(jnp.sum(k * k, axis=-1, keepdims=True)), inv_eps)
        qns.append(qn.astype(jnp.bfloat16))
        kns.append(kn.astype(jnp.bfloat16))

    row_head = lax.broadcasted_iota(jnp.int32, (dim, dim), 0) // c_head
    col_head = lax.broadcasted_iota(jnp.int32, (dim, dim), 1) // c_head
    offdiag = row_head != col_head
    ps = []
    for qn, kn in zip(qns, kns):
        gram = lax.dot_general(qn, kn, (((1,), (1,)), ((), ())),
                               preferred_element_type=jnp.float32)
        gram = gram * trow_ref[...]
        gram = jnp.where(offdiag, jnp.float32(-1e30), gram)
        gram = gram - jnp.max(gram, axis=-1, keepdims=True)
        p = jnp.exp(gram)
        p = p * pl.reciprocal(jnp.sum(p, axis=-1, keepdims=True), approx=True)
        ps.append(p.astype(jnp.bfloat16))

    outs = []
    for p, v in zip(ps, vs):
        ctx = jnp.dot(p, v.astype(jnp.bfloat16),
                      preferred_element_type=jnp.float32)
        outs.append(jnp.dot(projw_ref[...], ctx.astype(jnp.bfloat16),
                            preferred_element_type=jnp.float32))
    return outs


def _sgta_kernel(qkvw_ref, dww_ref, x_ref, projw_ref, trow_ref, o_ref,
                 w3_ref, *, nb, dim, num_heads, h, w):
    @pl.when(pl.program_id(0) == 0)
    def _():
        qkvw = qkvw_ref[...]
        for tap in range(9):
            w3_ref[:, tap * dim:(tap + 1) * dim] = (
                dww_ref[:, tap:tap + 1] * qkvw).astype(w3_ref.dtype)

    n = h * w
    for pb in range(nb // 2):
        x2 = jnp.concatenate(
            [x_ref[2 * pb].astype(jnp.bfloat16),
             x_ref[2 * pb + 1].astype(jnp.bfloat16)], axis=1)
        xstack2 = _prep_taps(x2, h=h, w=w)
        qkv2 = jnp.dot(w3_ref[...], xstack2,
                       preferred_element_type=jnp.float32)
        outs = _sgta_tails([qkv2[:, i * n:(i + 1) * n] for i in range(2)],
                           projw_ref, trow_ref, dim=dim, num_heads=num_heads)
        for i in range(2):
            o_ref[2 * pb + i] = outs[i].astype(o_ref.dtype)
    if nb % 2:
        qkv = jnp.dot(w3_ref[...], _prep_taps(x_ref[nb - 1].astype(jnp.bfloat16),
                                     h=h, w=w),
                      preferred_element_type=jnp.float32)
        outs = _sgta_tails([qkv], projw_ref, trow_ref,
                           dim=dim, num_heads=num_heads)
        o_ref[nb - 1] = outs[0].astype(o_ref.dtype)


def kernel(x, qkv_w, qkv_dw_w, proj_w, temperature):
    b, c, h, w = x.shape
    n = h * w
    num_heads = temperature.size
    c_head = c // num_heads
    c3 = 3 * c

    x_cn = x.reshape(b, c, n)

    dww = qkv_dw_w.reshape(c3, 9)
    trow = jnp.repeat(temperature.reshape(-1).astype(jnp.float32),
                      c_head).reshape(c, 1)

    nb = 4 if b % 4 == 0 else 1
    body = functools.partial(_sgta_kernel, nb=nb, dim=c,
                             num_heads=num_heads, h=h, w=w)
    out = pl.pallas_call(
        body,
        out_shape=jax.ShapeDtypeStruct((b, c, n), x.dtype),
        grid=(b // nb,),
        in_specs=[
            pl.BlockSpec((c3, c), lambda bi: (0, 0)),
            pl.BlockSpec((c3, 9), lambda bi: (0, 0)),
            pl.BlockSpec((nb, c, n), lambda bi: (bi, 0, 0)),
            pl.BlockSpec((c, c), lambda bi: (0, 0)),
            pl.BlockSpec((c, 1), lambda bi: (0, 0)),
        ],
        out_specs=pl.BlockSpec((nb, c, n), lambda bi: (bi, 0, 0)),
        scratch_shapes=[pltpu.VMEM((c3, 9 * c), jnp.bfloat16)],
        compiler_params=pltpu.CompilerParams(
            dimension_semantics=("arbitrary",),
            vmem_limit_bytes=_VMEM_LIMIT),
    )(qkv_w, dww, x_cn, proj_w.astype(jnp.bfloat16), trow)
    return out.reshape(b, c, h, w)
